# baseline (speedup 1.0000x reference)
import functools

import jax
import jax.numpy as jnp
from jax import lax
from jax.experimental import pallas as pl
from jax.experimental.pallas import tpu as pltpu
from jax.experimental.pallas import tpu_sc as plsc

_B = 4096
_L = 200
_D = 64
_H = 256

_INFO = plsc.get_sparse_core_info()
_NC = _INFO.num_cores
_NS = _INFO.num_subcores
_NW = _NC * _NS
_RPW = _B // _NW
_C0 = 128
_C1 = _L - _C0
_UNROLL = 8


def _pool_body(x_hbm, table_hbm, out_hbm, xv, rows, outv, sem):
    wid = lax.axis_index("s") * _NC + lax.axis_index("c")
    base = wid * _RPW

    pltpu.sync_copy(x_hbm.at[pl.ds(base, _RPW)], xv)

    inv_l = jnp.full((16,), 1.0 / _L, dtype=jnp.float32)

    def row_body(r, carry):
        cp0 = pltpu.async_copy(
            table_hbm.at[xv.at[r, pl.ds(0, _C0)]], rows.at[pl.ds(0, _C0)], sem)
        cp1 = pltpu.async_copy(
            table_hbm.at[xv.at[r, pl.ds(_C0, _C1)]], rows.at[pl.ds(_C0, _C1)],
            sem)
        cp0.wait()
        cp1.wait()

        def acc_body(j, acc):
            a0, a1, a2, a3 = acc
            for u in range(_UNROLL):
                jj = j * _UNROLL + u
                a0 = a0 + rows[jj, pl.ds(0, 16)]
                a1 = a1 + rows[jj, pl.ds(16, 16)]
                a2 = a2 + rows[jj, pl.ds(32, 16)]
                a3 = a3 + rows[jj, pl.ds(48, 16)]
            return (a0, a1, a2, a3)

        z = jnp.zeros((16,), dtype=jnp.float32)
        a0, a1, a2, a3 = lax.fori_loop(
            0, _L // _UNROLL, acc_body, (z, z, z, z))

        outv[r, pl.ds(0, 16)] = a0 * inv_l
        outv[r, pl.ds(16, 16)] = a1 * inv_l
        outv[r, pl.ds(32, 16)] = a2 * inv_l
        outv[r, pl.ds(48, 16)] = a3 * inv_l
        return carry

    lax.fori_loop(0, _RPW, row_body, 0)

    pltpu.sync_copy(outv, out_hbm.at[pl.ds(base, _RPW)])


@jax.jit
def _sc_pool(x, table):
    mesh = plsc.VectorSubcoreMesh(core_axis_name="c", subcore_axis_name="s")
    return pl.kernel(
        _pool_body,
        out_type=jax.ShapeDtypeStruct((_B, _D), jnp.float32),
        mesh=mesh,
        scratch_types=[
            pltpu.VMEM((_RPW, _L), jnp.int32),
            pltpu.VMEM((_L, _D), jnp.float32),
            pltpu.VMEM((_RPW, _D), jnp.float32),
            pltpu.SemaphoreType.DMA,
        ],
        compiler_params=pltpu.CompilerParams(use_tc_tiling_on_sc=False),
    )(x, table)


def _mlp_body(pooled_ref, w1_ref, b1_ref, w2_ref, b2_ref, out_ref):
    pooled = pooled_ref[...]
    hidden = lax.dot_general(
        pooled, w1_ref[...], (((1,), (1,)), ((), ())),
        preferred_element_type=jnp.float32)
    hidden = jnp.maximum(hidden + b1_ref[...], 0.0)
    out = jnp.sum(hidden * w2_ref[...], axis=1, keepdims=True)
    out_ref[...] = out + b2_ref[0]


@jax.jit
def _tc_mlp(pooled, W1, b1, W2, b2):
    out = pl.pallas_call(
        _mlp_body,
        in_specs=[
            pl.BlockSpec(memory_space=pltpu.VMEM),
            pl.BlockSpec(memory_space=pltpu.VMEM),
            pl.BlockSpec(memory_space=pltpu.VMEM),
            pl.BlockSpec(memory_space=pltpu.VMEM),
            pl.BlockSpec(memory_space=pltpu.SMEM),
        ],
        out_shape=jax.ShapeDtypeStruct((_B, 1), jnp.float32),
    )(pooled, W1, b1.reshape(1, _H), W2, b2)
    return jnp.squeeze(out, axis=-1)


def kernel(x, table, W1, b1, W2, b2):
    pooled = _sc_pool(x, table)
    return _tc_mlp(pooled, W1, b1, W2, b2)


# speedup vs baseline: 1.1983x; 1.1983x over previous
import functools

import jax
import jax.numpy as jnp
from jax import lax
from jax.experimental import pallas as pl
from jax.experimental.pallas import tpu as pltpu
from jax.experimental.pallas import tpu_sc as plsc

_B = 4096
_L = 200
_D = 64
_H = 256

_INFO = plsc.get_sparse_core_info()
_NC = _INFO.num_cores
_NS = _INFO.num_subcores
_NW = _NC * _NS
_RPW = _B // _NW
_C0 = 128
_C1 = _L - _C0
_UNROLL = 8


_NBUF = 4


def _pool_body(x_hbm, table_hbm, out_hbm, xv, rows0, rows1, rows2, rows3,
               outv, sem0, sem1, sem2, sem3):
    wid = lax.axis_index("s") * _NC + lax.axis_index("c")
    base = wid * _RPW
    bufs = (rows0, rows1, rows2, rows3)
    sems = (sem0, sem1, sem2, sem3)

    pltpu.sync_copy(x_hbm.at[pl.ds(base, _RPW)], xv)

    inv_l = jnp.full((16,), 1.0 / _L, dtype=jnp.float32)

    def _gather(r, buf, sem, issue):
        cp0 = pltpu.make_async_copy(
            table_hbm.at[xv.at[r, pl.ds(0, _C0)]], buf.at[pl.ds(0, _C0)], sem)
        cp1 = pltpu.make_async_copy(
            table_hbm.at[xv.at[r, pl.ds(_C0, _C1)]], buf.at[pl.ds(_C0, _C1)],
            sem)
        if issue:
            cp0.start()
            cp1.start()
        else:
            cp0.wait()
            cp1.wait()

    for r in range(_NBUF - 1):
        _gather(r, bufs[r], sems[r], issue=True)

    def iter_body(i, carry):
        for p in range(_NBUF):
            r = i * _NBUF + p
            nxt = r + (_NBUF - 1)

            @pl.when(nxt < _RPW)
            def _():
                _gather(nxt, bufs[(p + _NBUF - 1) % _NBUF],
                        sems[(p + _NBUF - 1) % _NBUF], issue=True)

            buf = bufs[p]
            _gather(r, buf, sems[p], issue=False)

            def acc_body(j, acc, buf=buf):
                a0, a1, a2, a3 = acc
                for u in range(_UNROLL):
                    jj = j * _UNROLL + u
                    a0 = a0 + buf[jj, pl.ds(0, 16)]
                    a1 = a1 + buf[jj, pl.ds(16, 16)]
                    a2 = a2 + buf[jj, pl.ds(32, 16)]
                    a3 = a3 + buf[jj, pl.ds(48, 16)]
                return (a0, a1, a2, a3)

            z = jnp.zeros((16,), dtype=jnp.float32)
            a0, a1, a2, a3 = lax.fori_loop(
                0, _L // _UNROLL, acc_body, (z, z, z, z))

            outv[r, pl.ds(0, 16)] = a0 * inv_l
            outv[r, pl.ds(16, 16)] = a1 * inv_l
            outv[r, pl.ds(32, 16)] = a2 * inv_l
            outv[r, pl.ds(48, 16)] = a3 * inv_l
        return carry

    lax.fori_loop(0, _RPW // _NBUF, iter_body, 0)

    pltpu.sync_copy(outv, out_hbm.at[pl.ds(base, _RPW)])


@jax.jit
def _sc_pool(x, table):
    mesh = plsc.VectorSubcoreMesh(core_axis_name="c", subcore_axis_name="s")
    return pl.kernel(
        _pool_body,
        out_type=jax.ShapeDtypeStruct((_B, _D), jnp.float32),
        mesh=mesh,
        scratch_types=[
            pltpu.VMEM((_RPW, _L), jnp.int32),
            pltpu.VMEM((_L, _D), jnp.float32),
            pltpu.VMEM((_L, _D), jnp.float32),
            pltpu.VMEM((_L, _D), jnp.float32),
            pltpu.VMEM((_L, _D), jnp.float32),
            pltpu.VMEM((_RPW, _D), jnp.float32),
            pltpu.SemaphoreType.DMA,
            pltpu.SemaphoreType.DMA,
            pltpu.SemaphoreType.DMA,
            pltpu.SemaphoreType.DMA,
        ],
        compiler_params=pltpu.CompilerParams(use_tc_tiling_on_sc=False),
    )(x, table)


def _mlp_body(pooled_ref, w1_ref, b1_ref, w2_ref, b2_ref, out_ref):
    pooled = pooled_ref[...]
    hidden = lax.dot_general(
        pooled, w1_ref[...], (((1,), (1,)), ((), ())),
        preferred_element_type=jnp.float32)
    hidden = jnp.maximum(hidden + b1_ref[...], 0.0)
    out = jnp.sum(hidden * w2_ref[...], axis=1, keepdims=True)
    out_ref[...] = out + b2_ref[0]


@jax.jit
def _tc_mlp(pooled, W1, b1, W2, b2):
    out = pl.pallas_call(
        _mlp_body,
        in_specs=[
            pl.BlockSpec(memory_space=pltpu.VMEM),
            pl.BlockSpec(memory_space=pltpu.VMEM),
            pl.BlockSpec(memory_space=pltpu.VMEM),
            pl.BlockSpec(memory_space=pltpu.VMEM),
            pl.BlockSpec(memory_space=pltpu.SMEM),
        ],
        out_shape=jax.ShapeDtypeStruct((_B, 1), jnp.float32),
    )(pooled, W1, b1.reshape(1, _H), W2, b2)
    return jnp.squeeze(out, axis=-1)


def kernel(x, table, W1, b1, W2, b2):
    pooled = _sc_pool(x, table)
    return _tc_mlp(pooled, W1, b1, W2, b2)
